# baseline (speedup 1.0000x reference)
import functools

import jax
import jax.numpy as jnp
from jax import lax
from jax.experimental import pallas as pl
from jax.experimental.pallas import tpu as pltpu
from jax.experimental.pallas import tpu_sc as plsc

N = 10000
E = 320000
T = 4
CH = 128

NC = 2
NS = 16
NW = NC * NS
EPW = E // NW
QI = 128
SLOTS = 10240
RPT8 = SLOTS // QI
DSLOTS = 40960
DRPT = DSLOTS // QI
NPAD = 10240
DUMMY = N
ZPT = NPAD // NS
DRW = 624
DW = 16

_f32 = jnp.float32
_mesh = plsc.VectorSubcoreMesh(core_axis_name="c", subcore_axis_name="s")


def _make_apply(nt, rpt8, nchunk):
    chunk = rpt8 // nchunk
    assert chunk % 8 == 0 and chunk <= 128

    @functools.partial(
        pl.kernel,
        out_type=jax.ShapeDtypeStruct((NC * nt * N, CH), _f32),
        mesh=_mesh,
        scratch_types=[
            pltpu.VMEM_SHARED((NPAD, CH), _f32),
            pltpu.VMEM((chunk, QI), jnp.int32),
            pltpu.VMEM((chunk, QI), jnp.int32),
            pltpu.VMEM((QI, CH), _f32),
            pltpu.VMEM((QI, CH), _f32),
            pltpu.SemaphoreType.DMA,
            pltpu.SemaphoreType.DMA,
        ],
    )
    def apply_k(tab, srcg, dstm, zc, out, acc, srcb, dstb, rows0, rows1, gsem, ssem):
        cid = lax.axis_index("c")
        sid = lax.axis_index("s")
        wid = cid * NS + sid
        bufs = (rows0, rows1)

        def per_t(t, carry):
            pltpu.sync_copy(zc, rows0)
            for c in range(ZPT // QI):
                pltpu.sync_copy(rows0, acc.at[pl.ds(sid * ZPT + c * QI, QI)])
            plsc.subcore_barrier()
            half = chunk
            for h in range(nchunk):
                slab = (t * NW + wid) * rpt8 + h * half
                pltpu.sync_copy(srcg.at[pl.ds(slab, half)], srcb)
                pltpu.sync_copy(dstm.at[pl.ds(slab, half)], dstb)
                gh = [None, None]
                sh = [None, None]
                gh[0] = pltpu.async_copy(tab.at[srcb.at[0]], rows0, gsem)
                for j in range(half):
                    b = j % 2
                    if j + 1 < half:
                        nb = (j + 1) % 2
                        if sh[nb] is not None:
                            sh[nb].wait()
                        gh[nb] = pltpu.async_copy(tab.at[srcb.at[j + 1]],
                                                  bufs[nb], gsem)
                    gh[b].wait()
                    sh[b] = pltpu.async_copy(bufs[b], acc.at[dstb.at[j]],
                                             ssem, add=True)
                sh[0].wait()
                sh[1].wait()
            plsc.subcore_barrier()
            outbase = cid * (nt * N) + t * N
            pltpu.sync_copy(acc.at[pl.ds(sid * DRW, DRW)],
                            out.at[pl.ds(outbase + sid * DRW, DRW)])

            @pl.when(sid == NS - 1)
            def _():
                pltpu.sync_copy(acc.at[pl.ds(NS * DRW, N - NS * DRW)],
                                out.at[pl.ds(outbase + NS * DRW, N - NS * DRW)])

            plsc.subcore_barrier()
            return carry

        if nt == 1:
            per_t(0, 0)
        else:
            lax.fori_loop(0, nt, per_t, 0)

    return apply_k


_apply1 = _make_apply(1, rpt8=RPT8, nchunk=10)
_apply4 = _make_apply(T, rpt8=RPT8, nchunk=10)
_applyd = _make_apply(1, rpt8=DRPT, nchunk=40)


def _prep_call(degp, oh, xf):
    BM = 2000
    NB = N // BM

    def body(dp, o, x, dv, y):
        deg = jnp.dot(dp[0] + dp[1], o[0], preferred_element_type=_f32)
        d = jnp.where(deg > 0, lax.rsqrt(jnp.where(deg > 0, deg, 1.0)), 0.0)
        dv[...] = d
        y[...] = x[...] * d

    return pl.pallas_call(
        body,
        grid=(T, NB),
        in_specs=[pl.BlockSpec((2, BM, CH), lambda t, i: (0, i, 0)),
                  pl.BlockSpec((1, CH, 1), lambda t, i: (t, 0, 0)),
                  pl.BlockSpec((BM, CH), lambda t, i: (t * NB + i, 0))],
        out_specs=[pl.BlockSpec((BM, 1), lambda t, i: (t * NB + i, 0)),
                   pl.BlockSpec((BM, CH), lambda t, i: (t * NB + i, 0))],
        out_shape=[jax.ShapeDtypeStruct((T * N, 1), _f32),
                   jax.ShapeDtypeStruct((T * N, CH), _f32)],
    )(degp, oh, xf)


def _scale_call(up, dinv, toff=0):
    M = up.shape[1]
    BM = 2000
    ob = toff * (N // BM)

    def body(u, dv, s_ref, v_ref):
        d = dv[...]
        s = -(d * (u[0] + u[1]))
        s_ref[...] = s
        v_ref[...] = d * s

    return pl.pallas_call(
        body,
        grid=(M // BM,),
        in_specs=[pl.BlockSpec((2, BM, CH), lambda i: (0, i, 0)),
                  pl.BlockSpec((BM, 1), lambda i: (i + ob, 0))],
        out_specs=[pl.BlockSpec((BM, CH), lambda i: (i, 0)),
                   pl.BlockSpec((BM, CH), lambda i: (i, 0))],
        out_shape=[jax.ShapeDtypeStruct((M, CH), _f32),
                   jax.ShapeDtypeStruct((M, CH), _f32)],
    )(up, dinv)


def _xmat_call(u2p, dinv, xf, sx, wc, bc):
    M = xf.shape[0]
    BM = 2000

    def body(u, dv, x, s, w, b, o):
        d = dv[...]
        xb = x[...]
        ssx = -(d * (u[0] + u[1]))
        t2 = 2.0 * ssx - xb
        acc = jnp.dot(xb, w[0], preferred_element_type=_f32)
        acc += jnp.dot(s[...], w[1], preferred_element_type=_f32)
        acc += jnp.dot(t2, w[2], preferred_element_type=_f32)
        o[...] = acc + b[...]

    return pl.pallas_call(
        body,
        grid=(M // BM,),
        in_specs=[pl.BlockSpec((2, BM, CH), lambda i: (0, i, 0)),
                  pl.BlockSpec((BM, 1), lambda i: (i, 0)),
                  pl.BlockSpec((BM, CH), lambda i: (i, 0)),
                  pl.BlockSpec((BM, CH), lambda i: (i, 0)),
                  pl.BlockSpec((3, CH, 3 * CH), lambda i: (0, 0, 0)),
                  pl.BlockSpec((1, 3 * CH), lambda i: (0, 0))],
        out_specs=pl.BlockSpec((BM, 3 * CH), lambda i: (i, 0)),
        out_shape=jax.ShapeDtypeStruct((M, 3 * CH), _f32),
    )(u2p, dinv, xf, sx, wc, bc)


def _gate_call(a2p, dinv, xc, h, sh, wzr, bzr, t):
    BM = 2000
    ob = t * (N // BM)

    def body(u, dv, xcb, hb, shb, w, b, z_ref, hr_ref, w_ref):
        d = dv[...]
        hh = hb[...]
        ssh = -(d * (u[0] + u[1]))
        t2 = 2.0 * ssh - hh
        acc = jnp.dot(hh, w[0], preferred_element_type=_f32)
        acc += jnp.dot(shb[...], w[1], preferred_element_type=_f32)
        acc += jnp.dot(t2, w[2], preferred_element_type=_f32)
        g = jax.nn.sigmoid(xcb[...] + acc + b[...])
        z = g[:, :CH]
        hr = hh * g[:, CH:]
        z_ref[...] = z
        hr_ref[...] = hr
        w_ref[...] = d * hr

    return pl.pallas_call(
        body,
        grid=(N // BM,),
        in_specs=[pl.BlockSpec((2, BM, CH), lambda i: (0, i, 0)),
                  pl.BlockSpec((BM, 1), lambda i: (i + ob, 0)),
                  pl.BlockSpec((BM, 2 * CH), lambda i: (i + ob, 0)),
                  pl.BlockSpec((BM, CH), lambda i: (i, 0)),
                  pl.BlockSpec((BM, CH), lambda i: (i, 0)),
                  pl.BlockSpec((3, CH, 2 * CH), lambda i: (0, 0, 0)),
                  pl.BlockSpec((1, 2 * CH), lambda i: (0, 0))],
        out_specs=[pl.BlockSpec((BM, CH), lambda i: (i, 0))] * 3,
        out_shape=[jax.ShapeDtypeStruct((N, CH), _f32)] * 3,
    )(a2p, dinv, xc, h, sh, wzr, bzr)


def _update_call(b2p, dinv, xc, hr, shr, whh, bhh, z, h, t):
    BM = 2000
    ob = t * (N // BM)
    obn = min(t + 1, T - 1) * (N // BM)

    def body(u, dv, dvn, xcb, hrb, shrb, w, b, zb, hb, hn_ref, hd_ref):
        d = dv[...]
        hr_ = hrb[...]
        ss = -(d * (u[0] + u[1]))
        t2 = 2.0 * ss - hr_
        acc = jnp.dot(hr_, w[0], preferred_element_type=_f32)
        acc += jnp.dot(shrb[...], w[1], preferred_element_type=_f32)
        acc += jnp.dot(t2, w[2], preferred_element_type=_f32)
        ht = jnp.tanh(xcb[...] + acc + b[...])
        zz = zb[...]
        hn = zz * hb[...] + (1.0 - zz) * ht
        hn_ref[...] = hn
        hd_ref[...] = dvn[...] * hn

    return pl.pallas_call(
        body,
        grid=(N // BM,),
        in_specs=[pl.BlockSpec((2, BM, CH), lambda i: (0, i, 0)),
                  pl.BlockSpec((BM, 1), lambda i: (i + ob, 0)),
                  pl.BlockSpec((BM, 1), lambda i: (i + obn, 0)),
                  pl.BlockSpec((BM, CH), lambda i: (i + ob, 2)),
                  pl.BlockSpec((BM, CH), lambda i: (i, 0)),
                  pl.BlockSpec((BM, CH), lambda i: (i, 0)),
                  pl.BlockSpec((3, CH, CH), lambda i: (0, 0, 0)),
                  pl.BlockSpec((1, CH), lambda i: (0, 0)),
                  pl.BlockSpec((BM, CH), lambda i: (i, 0)),
                  pl.BlockSpec((BM, CH), lambda i: (i, 0))],
        out_specs=[pl.BlockSpec((BM, CH), lambda i: (i, 0))] * 2,
        out_shape=[jax.ShapeDtypeStruct((N, CH), _f32)] * 2,
    )(b2p, dinv, dinv, xc, hr, shr, whh, bhh, z, h)


def _proj_call(h, wp, bp):
    BM = 2000

    def body(hb, w, b, o):
        o[...] = jnp.dot(hb[...], w[...], preferred_element_type=_f32) + b[...]

    return pl.pallas_call(
        body,
        grid=(N // BM,),
        in_specs=[pl.BlockSpec((BM, CH), lambda i: (i, 0)),
                  pl.BlockSpec((CH, CH), lambda i: (0, 0)),
                  pl.BlockSpec((1, CH), lambda i: (0, 0))],
        out_specs=pl.BlockSpec((BM, CH), lambda i: (i, 0)),
        out_shape=jax.ShapeDtypeStruct((N, CH), _f32),
    )(h, wp, bp)


def kernel(features_seq, edges_seq, Wx, bx, Wh, bh, Wp, bp):
    i32 = jnp.int32
    src = edges_seq[:, 0]
    dst = edges_seq[:, 1]
    self_m = src == dst
    srcm = jnp.where(self_m, DUMMY, src).astype(i32)
    dstm = jnp.where(self_m, DUMMY, dst).astype(i32)
    toff = (jnp.arange(T, dtype=i32) * N)[:, None]
    srcx = (src + toff).astype(i32)

    def pad_idx(a, fill):
        a4 = a.reshape(T, NW, EPW)
        padcols = jnp.full((T, NW, SLOTS - EPW), fill, i32)
        return jnp.concatenate([a4, padcols], axis=2).reshape(T * NW * RPT8, QI)

    def pad_idx_d(a, fill):
        a3 = a.reshape(NW, T * E // NW)
        padcols = jnp.full((NW, DSLOTS - T * E // NW), fill, i32)
        return jnp.concatenate([a3, padcols], axis=1).reshape(NW * DRPT, QI)

    dstm2 = pad_idx(dstm, DUMMY)
    srcx2 = pad_idx(srcx, 0)
    srcr = pad_idx(src, 0).reshape(T, NW * RPT8, QI)
    dstmr = dstm2.reshape(T, NW * RPT8, QI)
    srcxd = pad_idx_d(srcx, 0)
    srcmd = pad_idx_d(srcm, DUMMY)

    zc128 = jnp.zeros((QI, CH), _f32)

    wc = jnp.transpose(Wx, (1, 2, 0, 3)).reshape(3, CH, 3 * CH)
    bc = bx.reshape(1, 3 * CH)
    wzr = jnp.transpose(Wh[:2], (1, 2, 0, 3)).reshape(3, CH, 2 * CH)
    bzr = bh[:2].reshape(1, 2 * CH)
    whh = Wh[2]
    bhh = bh[2].reshape(1, CH)
    bp2 = bp.reshape(1, CH)

    xf = features_seq.reshape(T * N, CH)

    pat = jnp.repeat(jnp.eye(T, dtype=_f32), CH // T, axis=1)
    tabd = jnp.broadcast_to(pat[:, None, :], (T, N, CH)).reshape(T * N, CH)
    oh = jnp.eye(CH, dtype=_f32)[(CH // T) * jnp.arange(T)][:, :, None]
    degp = _applyd(tabd, srcxd, srcmd, zc128).reshape(NC, N, CH)
    dinv, yhat = _prep_call(degp, oh, xf)

    u1 = _apply4(yhat, srcx2, dstm2, zc128).reshape(NC, T * N, CH)
    sx, v2 = _scale_call(u1, dinv)
    u2 = _apply4(v2, srcx2, dstm2, zc128).reshape(NC, T * N, CH)
    xc = _xmat_call(u2, dinv, xf, sx, wc, bc)

    zN = jnp.zeros((N, CH), _f32)
    z2 = jnp.zeros((NC, N, CH), _f32)
    H = zN
    Hd = zN
    for t in range(T):
        if t == 0:
            z, hr, wv = _gate_call(z2, dinv, xc, zN, zN, wzr, bzr, t)
            hn, hd = _update_call(z2, dinv, xc, zN, zN, whh, bhh, z, zN, t)
        else:
            a1 = _apply1(Hd, srcr[t], dstmr[t], zc128).reshape(NC, N, CH)
            sh, va = _scale_call(a1, dinv, toff=t)
            a2 = _apply1(va, srcr[t], dstmr[t], zc128).reshape(NC, N, CH)
            z, hr, wv = _gate_call(a2, dinv, xc, H, sh, wzr, bzr, t)
            b1 = _apply1(wv, srcr[t], dstmr[t], zc128).reshape(NC, N, CH)
            shr, vb = _scale_call(b1, dinv, toff=t)
            b2 = _apply1(vb, srcr[t], dstmr[t], zc128).reshape(NC, N, CH)
            hn, hd = _update_call(b2, dinv, xc, hr, shr, whh, bhh, z, H, t)
        H = hn
        Hd = hd
    return _proj_call(H, Wp, bp2)


# speedup vs baseline: 1.1538x; 1.1538x over previous
import functools

import jax
import jax.numpy as jnp
from jax import lax
from jax.experimental import pallas as pl
from jax.experimental.pallas import tpu as pltpu
from jax.experimental.pallas import tpu_sc as plsc

N = 10000
E = 320000
T = 4
CH = 128

NC = 2
NS = 16
NW = NC * NS
EPW = E // NW
QI = 128
SLOTS = 10240
RPT8 = SLOTS // QI
DSLOTS = 40960
DRPT = DSLOTS // QI
NPAD = 10240
DUMMY = N
ZPT = NPAD // NS
DRW = 624
DW = 16

_f32 = jnp.float32
_mesh = plsc.VectorSubcoreMesh(core_axis_name="c", subcore_axis_name="s")


def _make_apply(nt, rpt8, nchunk):
    chunk = rpt8 // nchunk
    assert chunk % 8 == 0 and chunk <= 128

    @functools.partial(
        pl.kernel,
        out_type=jax.ShapeDtypeStruct((NC * nt * N, CH), _f32),
        mesh=_mesh,
        scratch_types=[
            pltpu.VMEM_SHARED((NPAD, CH), _f32),
            pltpu.VMEM((chunk, QI), jnp.int32),
            pltpu.VMEM((chunk, QI), jnp.int32),
            pltpu.VMEM((QI, CH), _f32),
            pltpu.VMEM((QI, CH), _f32),
            pltpu.SemaphoreType.DMA,
            pltpu.SemaphoreType.DMA,
        ],
    )
    def apply_k(tab, srcg, dstm, zc, out, acc, srcb, dstb, rows0, rows1, gsem, ssem):
        cid = lax.axis_index("c")
        sid = lax.axis_index("s")
        wid = cid * NS + sid
        bufs = (rows0, rows1)

        def per_t(t, carry):
            pltpu.sync_copy(zc, rows0)
            for c in range(ZPT // QI):
                pltpu.sync_copy(rows0, acc.at[pl.ds(sid * ZPT + c * QI, QI)])
            plsc.subcore_barrier()
            half = chunk
            for h in range(nchunk):
                slab = (t * NW + wid) * rpt8 + h * half
                pltpu.sync_copy(srcg.at[pl.ds(slab, half)], srcb)
                pltpu.sync_copy(dstm.at[pl.ds(slab, half)], dstb)
                gh = [None, None]
                sh = [None, None]
                gh[0] = pltpu.async_copy(tab.at[srcb.at[0]], rows0, gsem)
                for j in range(half):
                    b = j % 2
                    if j + 1 < half:
                        nb = (j + 1) % 2
                        if sh[nb] is not None:
                            sh[nb].wait()
                        gh[nb] = pltpu.async_copy(tab.at[srcb.at[j + 1]],
                                                  bufs[nb], gsem)
                    gh[b].wait()
                    sh[b] = pltpu.async_copy(bufs[b], acc.at[dstb.at[j]],
                                             ssem, add=True)
                sh[0].wait()
                sh[1].wait()
            plsc.subcore_barrier()
            outbase = cid * (nt * N) + t * N
            pltpu.sync_copy(acc.at[pl.ds(sid * DRW, DRW)],
                            out.at[pl.ds(outbase + sid * DRW, DRW)])

            @pl.when(sid == NS - 1)
            def _():
                pltpu.sync_copy(acc.at[pl.ds(NS * DRW, N - NS * DRW)],
                                out.at[pl.ds(outbase + NS * DRW, N - NS * DRW)])

            plsc.subcore_barrier()
            return carry

        if nt == 1:
            per_t(0, 0)
        else:
            lax.fori_loop(0, nt, per_t, 0)

    return apply_k


_apply1 = _make_apply(1, rpt8=RPT8, nchunk=10)
_apply4 = _make_apply(T, rpt8=RPT8, nchunk=10)


@functools.partial(
    pl.kernel,
    out_type=jax.ShapeDtypeStruct((NC * N, CH), _f32),
    mesh=_mesh,
    scratch_types=[
        pltpu.VMEM_SHARED((NPAD, CH), _f32),
        pltpu.VMEM((16, QI), jnp.int32),
        pltpu.VMEM((16, QI), jnp.int32),
        pltpu.VMEM((QI, CH), _f32),
        pltpu.SemaphoreType.DMA,
    ],
)
def _deg_call(patt, dstm, zc, out, acc, dstb0, dstb1, rows, ssem):
    cid = lax.axis_index("c")
    sid = lax.axis_index("s")
    wid = cid * NS + sid
    pltpu.sync_copy(zc, rows)
    for c in range(ZPT // QI):
        pltpu.sync_copy(rows, acc.at[pl.ds(sid * ZPT + c * QI, QI)])
    plsc.subcore_barrier()
    dstbs = (dstb0, dstb1)
    pending = [[], []]
    for t in range(T):
        for b in (0, 1):
            for r in pending[b]:
                r.wait()
            pending[b] = []
        pltpu.sync_copy(patt.at[pl.ds(t * QI, QI)], rows)
        for h in range(RPT8 // 16):
            b = h % 2
            for r in pending[b]:
                r.wait()
            pending[b] = []
            pltpu.sync_copy(dstm.at[pl.ds((wid * T + t) * RPT8 + h * 16, 16)],
                            dstbs[b])
            for j in range(16):
                pending[b].append(pltpu.async_copy(rows, acc.at[dstbs[b].at[j]],
                                                   ssem, add=True))
    for b in (0, 1):
        for r in pending[b]:
            r.wait()
    plsc.subcore_barrier()
    outbase = cid * N
    pltpu.sync_copy(acc.at[pl.ds(sid * DRW, DRW)],
                    out.at[pl.ds(outbase + sid * DRW, DRW)])

    @pl.when(sid == NS - 1)
    def _():
        pltpu.sync_copy(acc.at[pl.ds(NS * DRW, N - NS * DRW)],
                        out.at[pl.ds(outbase + NS * DRW, N - NS * DRW)])


def _prep_call(degp, oh, xf):
    BM = 2000
    NB = N // BM

    def body(dp, o, x, dv, y):
        deg = jnp.dot(dp[0] + dp[1], o[0], preferred_element_type=_f32)
        d = jnp.where(deg > 0, lax.rsqrt(jnp.where(deg > 0, deg, 1.0)), 0.0)
        dv[...] = d
        y[...] = x[...] * d

    return pl.pallas_call(
        body,
        grid=(T, NB),
        in_specs=[pl.BlockSpec((2, BM, CH), lambda t, i: (0, i, 0)),
                  pl.BlockSpec((1, CH, 1), lambda t, i: (t, 0, 0)),
                  pl.BlockSpec((BM, CH), lambda t, i: (t * NB + i, 0))],
        out_specs=[pl.BlockSpec((BM, 1), lambda t, i: (t * NB + i, 0)),
                   pl.BlockSpec((BM, CH), lambda t, i: (t * NB + i, 0))],
        out_shape=[jax.ShapeDtypeStruct((T * N, 1), _f32),
                   jax.ShapeDtypeStruct((T * N, CH), _f32)],
    )(degp, oh, xf)


def _scale_call(up, dinv, toff=0):
    M = up.shape[1]
    BM = 2000
    ob = toff * (N // BM)

    def body(u, dv, s_ref, v_ref):
        d = dv[...]
        s = -(d * (u[0] + u[1]))
        s_ref[...] = s
        v_ref[...] = d * s

    return pl.pallas_call(
        body,
        grid=(M // BM,),
        in_specs=[pl.BlockSpec((2, BM, CH), lambda i: (0, i, 0)),
                  pl.BlockSpec((BM, 1), lambda i: (i + ob, 0))],
        out_specs=[pl.BlockSpec((BM, CH), lambda i: (i, 0)),
                   pl.BlockSpec((BM, CH), lambda i: (i, 0))],
        out_shape=[jax.ShapeDtypeStruct((M, CH), _f32),
                   jax.ShapeDtypeStruct((M, CH), _f32)],
    )(up, dinv)


def _xmat_call(u2p, dinv, xf, sx, wc, bc):
    M = xf.shape[0]
    BM = 2000

    def body(u, dv, x, s, w, b, o):
        d = dv[...]
        xb = x[...]
        ssx = -(d * (u[0] + u[1]))
        t2 = 2.0 * ssx - xb
        acc = jnp.dot(xb, w[0], preferred_element_type=_f32)
        acc += jnp.dot(s[...], w[1], preferred_element_type=_f32)
        acc += jnp.dot(t2, w[2], preferred_element_type=_f32)
        o[...] = acc + b[...]

    return pl.pallas_call(
        body,
        grid=(M // BM,),
        in_specs=[pl.BlockSpec((2, BM, CH), lambda i: (0, i, 0)),
                  pl.BlockSpec((BM, 1), lambda i: (i, 0)),
                  pl.BlockSpec((BM, CH), lambda i: (i, 0)),
                  pl.BlockSpec((BM, CH), lambda i: (i, 0)),
                  pl.BlockSpec((3, CH, 3 * CH), lambda i: (0, 0, 0)),
                  pl.BlockSpec((1, 3 * CH), lambda i: (0, 0))],
        out_specs=pl.BlockSpec((BM, 3 * CH), lambda i: (i, 0)),
        out_shape=jax.ShapeDtypeStruct((M, 3 * CH), _f32),
    )(u2p, dinv, xf, sx, wc, bc)


def _gate_call(a2p, dinv, xc, h, sh, wzr, bzr, t):
    BM = 2000
    ob = t * (N // BM)

    def body(u, dv, xcb, hb, shb, w, b, z_ref, hr_ref, w_ref):
        d = dv[...]
        hh = hb[...]
        ssh = -(d * (u[0] + u[1]))
        t2 = 2.0 * ssh - hh
        acc = jnp.dot(hh, w[0], preferred_element_type=_f32)
        acc += jnp.dot(shb[...], w[1], preferred_element_type=_f32)
        acc += jnp.dot(t2, w[2], preferred_element_type=_f32)
        g = jax.nn.sigmoid(xcb[...] + acc + b[...])
        z = g[:, :CH]
        hr = hh * g[:, CH:]
        z_ref[...] = z
        hr_ref[...] = hr
        w_ref[...] = d * hr

    return pl.pallas_call(
        body,
        grid=(N // BM,),
        in_specs=[pl.BlockSpec((2, BM, CH), lambda i: (0, i, 0)),
                  pl.BlockSpec((BM, 1), lambda i: (i + ob, 0)),
                  pl.BlockSpec((BM, 2 * CH), lambda i: (i + ob, 0)),
                  pl.BlockSpec((BM, CH), lambda i: (i, 0)),
                  pl.BlockSpec((BM, CH), lambda i: (i, 0)),
                  pl.BlockSpec((3, CH, 2 * CH), lambda i: (0, 0, 0)),
                  pl.BlockSpec((1, 2 * CH), lambda i: (0, 0))],
        out_specs=[pl.BlockSpec((BM, CH), lambda i: (i, 0))] * 3,
        out_shape=[jax.ShapeDtypeStruct((N, CH), _f32)] * 3,
    )(a2p, dinv, xc, h, sh, wzr, bzr)


def _update_call(b2p, dinv, xc, hr, shr, whh, bhh, z, h, t):
    BM = 2000
    ob = t * (N // BM)
    obn = min(t + 1, T - 1) * (N // BM)

    def body(u, dv, dvn, xcb, hrb, shrb, w, b, zb, hb, hn_ref, hd_ref):
        d = dv[...]
        hr_ = hrb[...]
        ss = -(d * (u[0] + u[1]))
        t2 = 2.0 * ss - hr_
        acc = jnp.dot(hr_, w[0], preferred_element_type=_f32)
        acc += jnp.dot(shrb[...], w[1], preferred_element_type=_f32)
        acc += jnp.dot(t2, w[2], preferred_element_type=_f32)
        ht = jnp.tanh(xcb[...] + acc + b[...])
        zz = zb[...]
        hn = zz * hb[...] + (1.0 - zz) * ht
        hn_ref[...] = hn
        hd_ref[...] = dvn[...] * hn

    return pl.pallas_call(
        body,
        grid=(N // BM,),
        in_specs=[pl.BlockSpec((2, BM, CH), lambda i: (0, i, 0)),
                  pl.BlockSpec((BM, 1), lambda i: (i + ob, 0)),
                  pl.BlockSpec((BM, 1), lambda i: (i + obn, 0)),
                  pl.BlockSpec((BM, CH), lambda i: (i + ob, 2)),
                  pl.BlockSpec((BM, CH), lambda i: (i, 0)),
                  pl.BlockSpec((BM, CH), lambda i: (i, 0)),
                  pl.BlockSpec((3, CH, CH), lambda i: (0, 0, 0)),
                  pl.BlockSpec((1, CH), lambda i: (0, 0)),
                  pl.BlockSpec((BM, CH), lambda i: (i, 0)),
                  pl.BlockSpec((BM, CH), lambda i: (i, 0))],
        out_specs=[pl.BlockSpec((BM, CH), lambda i: (i, 0))] * 2,
        out_shape=[jax.ShapeDtypeStruct((N, CH), _f32)] * 2,
    )(b2p, dinv, dinv, xc, hr, shr, whh, bhh, z, h)


def _proj_call(h, wp, bp):
    BM = 2000

    def body(hb, w, b, o):
        o[...] = jnp.dot(hb[...], w[...], preferred_element_type=_f32) + b[...]

    return pl.pallas_call(
        body,
        grid=(N // BM,),
        in_specs=[pl.BlockSpec((BM, CH), lambda i: (i, 0)),
                  pl.BlockSpec((CH, CH), lambda i: (0, 0)),
                  pl.BlockSpec((1, CH), lambda i: (0, 0))],
        out_specs=pl.BlockSpec((BM, CH), lambda i: (i, 0)),
        out_shape=jax.ShapeDtypeStruct((N, CH), _f32),
    )(h, wp, bp)


def kernel(features_seq, edges_seq, Wx, bx, Wh, bh, Wp, bp):
    i32 = jnp.int32
    src = edges_seq[:, 0]
    dst = edges_seq[:, 1]
    self_m = src == dst
    srcm = jnp.where(self_m, DUMMY, src).astype(i32)
    dstm = jnp.where(self_m, DUMMY, dst).astype(i32)
    toff = (jnp.arange(T, dtype=i32) * N)[:, None]
    srcx = (src + toff).astype(i32)

    def pad_idx(a, fill):
        a4 = a.reshape(T, NW, EPW)
        padcols = jnp.full((T, NW, SLOTS - EPW), fill, i32)
        return jnp.concatenate([a4, padcols], axis=2).reshape(T * NW * RPT8, QI)

    dstm2 = pad_idx(dstm, DUMMY)
    srcx2 = pad_idx(srcx, 0)
    srcr = pad_idx(src, 0).reshape(T, NW * RPT8, QI)
    dstmr = dstm2.reshape(T, NW * RPT8, QI)
    srcmd = jnp.transpose(pad_idx(srcm, DUMMY).reshape(T, NW, RPT8, QI),
                          (1, 0, 2, 3)).reshape(NW * T * RPT8, QI)

    zc128 = jnp.zeros((QI, CH), _f32)

    wc = jnp.transpose(Wx, (1, 2, 0, 3)).reshape(3, CH, 3 * CH)
    bc = bx.reshape(1, 3 * CH)
    wzr = jnp.transpose(Wh[:2], (1, 2, 0, 3)).reshape(3, CH, 2 * CH)
    bzr = bh[:2].reshape(1, 2 * CH)
    whh = Wh[2]
    bhh = bh[2].reshape(1, CH)
    bp2 = bp.reshape(1, CH)

    xf = features_seq.reshape(T * N, CH)

    pat = jnp.repeat(jnp.eye(T, dtype=_f32), CH // T, axis=1)
    patt = jnp.broadcast_to(pat[:, None, :], (T, QI, CH)).reshape(T * QI, CH)
    oh = jnp.eye(CH, dtype=_f32)[(CH // T) * jnp.arange(T)][:, :, None]
    degp = _deg_call(patt, srcmd, zc128).reshape(NC, N, CH)
    dinv, yhat = _prep_call(degp, oh, xf)

    u1 = _apply4(yhat, srcx2, dstm2, zc128).reshape(NC, T * N, CH)
    sx, v2 = _scale_call(u1, dinv)
    u2 = _apply4(v2, srcx2, dstm2, zc128).reshape(NC, T * N, CH)
    xc = _xmat_call(u2, dinv, xf, sx, wc, bc)

    zN = jnp.zeros((N, CH), _f32)
    z2 = jnp.zeros((NC, N, CH), _f32)
    H = zN
    Hd = zN
    for t in range(T):
        if t == 0:
            z, hr, wv = _gate_call(z2, dinv, xc, zN, zN, wzr, bzr, t)
            hn, hd = _update_call(z2, dinv, xc, zN, zN, whh, bhh, z, zN, t)
        else:
            a1 = _apply1(Hd, srcr[t], dstmr[t], zc128).reshape(NC, N, CH)
            sh, va = _scale_call(a1, dinv, toff=t)
            a2 = _apply1(va, srcr[t], dstmr[t], zc128).reshape(NC, N, CH)
            z, hr, wv = _gate_call(a2, dinv, xc, H, sh, wzr, bzr, t)
            b1 = _apply1(wv, srcr[t], dstmr[t], zc128).reshape(NC, N, CH)
            shr, vb = _scale_call(b1, dinv, toff=t)
            b2 = _apply1(vb, srcr[t], dstmr[t], zc128).reshape(NC, N, CH)
            hn, hd = _update_call(b2, dinv, xc, hr, shr, whh, bhh, z, H, t)
        H = hn
        Hd = hd
    return _proj_call(H, Wp, bp2)


# speedup vs baseline: 1.1830x; 1.0253x over previous
import functools

import jax
import jax.numpy as jnp
from jax import lax
from jax.experimental import pallas as pl
from jax.experimental.pallas import tpu as pltpu
from jax.experimental.pallas import tpu_sc as plsc

N = 10000
E = 320000
T = 4
CH = 128

NC = 2
NS = 16
NW = NC * NS
EPW = E // NW
QI = 64
SLOTS = 10240
RPT8 = SLOTS // QI
NPAD = 10240
DUMMY = N
ZPT = NPAD // NS
DRW = 624
DW = 16

_f32 = jnp.float32
_mesh = plsc.VectorSubcoreMesh(core_axis_name="c", subcore_axis_name="s")


def _make_apply(nt, rpt8, K=4, L=2, CHK=8):
    nchunk = rpt8 // CHK

    @functools.partial(
        pl.kernel,
        out_type=jax.ShapeDtypeStruct((NC * nt * N, CH), _f32),
        mesh=_mesh,
        scratch_types=[
            pltpu.VMEM_SHARED((NPAD, CH), _f32),
            pltpu.VMEM((CHK, QI), jnp.int32),
            pltpu.VMEM((CHK, QI), jnp.int32),
            pltpu.VMEM((CHK, QI), jnp.int32),
            pltpu.VMEM((CHK, QI), jnp.int32),
            pltpu.VMEM((QI, CH), _f32),
            pltpu.VMEM((QI, CH), _f32),
            pltpu.VMEM((QI, CH), _f32),
            pltpu.VMEM((QI, CH), _f32),
            pltpu.SemaphoreType.DMA,
            pltpu.SemaphoreType.DMA,
        ],
    )
    def apply_k(tab, srcg, dstm, zc, out, acc,
                srcb0, srcb1, dstb0, dstb1, r0, r1, r2, r3, gsem, ssem):
        cid = lax.axis_index("c")
        sid = lax.axis_index("s")
        wid = cid * NS + sid
        rows = (r0, r1, r2, r3)
        srcbs = (srcb0, srcb1)
        dstbs = (dstb0, dstb1)

        def scat(jj, gh, sh):
            k2 = jj % K
            gh[k2].wait()
            sh[k2] = pltpu.async_copy(
                rows[k2], acc.at[dstbs[(jj // CHK) % 2].at[jj % CHK]],
                ssem, add=True)

        def per_t(t, carry):
            pltpu.sync_copy(zc, r0)
            for c in range(ZPT // QI):
                pltpu.sync_copy(r0, acc.at[pl.ds(sid * ZPT + c * QI, QI)])
            plsc.subcore_barrier()
            gh = [None] * K
            sh = [None] * K
            for h in range(nchunk):
                b = h % 2
                slab = (t * NW + wid) * rpt8 + h * CHK
                pltpu.sync_copy(srcg.at[pl.ds(slab, CHK)], srcbs[b])
                pltpu.sync_copy(dstm.at[pl.ds(slab, CHK)], dstbs[b])
                for j8 in range(CHK):
                    j = h * CHK + j8
                    k = j % K
                    if sh[k] is not None:
                        sh[k].wait()
                    gh[k] = pltpu.async_copy(tab.at[srcbs[b].at[j8]],
                                             rows[k], gsem)
                    if j - L >= 0:
                        scat(j - L, gh, sh)
            for jj in range(rpt8 - L, rpt8):
                scat(jj, gh, sh)
            for k in range(K):
                if sh[k] is not None:
                    sh[k].wait()
            plsc.subcore_barrier()
            outbase = cid * (nt * N) + t * N
            pltpu.sync_copy(acc.at[pl.ds(sid * DRW, DRW)],
                            out.at[pl.ds(outbase + sid * DRW, DRW)])

            @pl.when(sid == NS - 1)
            def _():
                pltpu.sync_copy(acc.at[pl.ds(NS * DRW, N - NS * DRW)],
                                out.at[pl.ds(outbase + NS * DRW, N - NS * DRW)])

            plsc.subcore_barrier()
            return carry

        if nt == 1:
            per_t(0, 0)
        else:
            lax.fori_loop(0, nt, per_t, 0)

    return apply_k


_apply1 = _make_apply(1, rpt8=RPT8)
_apply4 = _make_apply(T, rpt8=RPT8)


@functools.partial(
    pl.kernel,
    out_type=jax.ShapeDtypeStruct((NC * N, CH), _f32),
    mesh=_mesh,
    scratch_types=[
        pltpu.VMEM_SHARED((NPAD, CH), _f32),
        pltpu.VMEM((16, QI), jnp.int32),
        pltpu.VMEM((16, QI), jnp.int32),
        pltpu.VMEM((QI, CH), _f32),
        pltpu.SemaphoreType.DMA,
    ],
)
def _deg_call(patt, dstm, zc, out, acc, dstb0, dstb1, rows, ssem):
    cid = lax.axis_index("c")
    sid = lax.axis_index("s")
    wid = cid * NS + sid
    pltpu.sync_copy(zc, rows)
    for c in range(ZPT // QI):
        pltpu.sync_copy(rows, acc.at[pl.ds(sid * ZPT + c * QI, QI)])
    plsc.subcore_barrier()
    dstbs = (dstb0, dstb1)
    pending = [[], []]
    for t in range(T):
        for b in (0, 1):
            for r in pending[b]:
                r.wait()
            pending[b] = []
        pltpu.sync_copy(patt.at[pl.ds(t * QI, QI)], rows)
        for h in range(RPT8 // 16):
            b = h % 2
            for r in pending[b]:
                r.wait()
            pending[b] = []
            pltpu.sync_copy(dstm.at[pl.ds((wid * T + t) * RPT8 + h * 16, 16)],
                            dstbs[b])
            for j in range(16):
                pending[b].append(pltpu.async_copy(rows, acc.at[dstbs[b].at[j]],
                                                   ssem, add=True))
    for b in (0, 1):
        for r in pending[b]:
            r.wait()
    plsc.subcore_barrier()
    outbase = cid * N
    pltpu.sync_copy(acc.at[pl.ds(sid * DRW, DRW)],
                    out.at[pl.ds(outbase + sid * DRW, DRW)])

    @pl.when(sid == NS - 1)
    def _():
        pltpu.sync_copy(acc.at[pl.ds(NS * DRW, N - NS * DRW)],
                        out.at[pl.ds(outbase + NS * DRW, N - NS * DRW)])


def _prep_call(degp, oh, xf):
    BM = 2000
    NB = N // BM

    def body(dp, o, x, dv, y):
        deg = jnp.dot(dp[0] + dp[1], o[0], preferred_element_type=_f32)
        d = jnp.where(deg > 0, lax.rsqrt(jnp.where(deg > 0, deg, 1.0)), 0.0)
        dv[...] = d
        y[...] = x[...] * d

    return pl.pallas_call(
        body,
        grid=(T, NB),
        in_specs=[pl.BlockSpec((2, BM, CH), lambda t, i: (0, i, 0)),
                  pl.BlockSpec((1, CH, 1), lambda t, i: (t, 0, 0)),
                  pl.BlockSpec((BM, CH), lambda t, i: (t * NB + i, 0))],
        out_specs=[pl.BlockSpec((BM, 1), lambda t, i: (t * NB + i, 0)),
                   pl.BlockSpec((BM, CH), lambda t, i: (t * NB + i, 0))],
        out_shape=[jax.ShapeDtypeStruct((T * N, 1), _f32),
                   jax.ShapeDtypeStruct((T * N, CH), _f32)],
    )(degp, oh, xf)


def _scale_call(up, dinv, toff=0):
    M = up.shape[1]
    BM = 2000
    ob = toff * (N // BM)

    def body(u, dv, s_ref, v_ref):
        d = dv[...]
        s = -(d * (u[0] + u[1]))
        s_ref[...] = s
        v_ref[...] = d * s

    return pl.pallas_call(
        body,
        grid=(M // BM,),
        in_specs=[pl.BlockSpec((2, BM, CH), lambda i: (0, i, 0)),
                  pl.BlockSpec((BM, 1), lambda i: (i + ob, 0))],
        out_specs=[pl.BlockSpec((BM, CH), lambda i: (i, 0)),
                   pl.BlockSpec((BM, CH), lambda i: (i, 0))],
        out_shape=[jax.ShapeDtypeStruct((M, CH), _f32),
                   jax.ShapeDtypeStruct((M, CH), _f32)],
    )(up, dinv)


def _xmat_call(u2p, dinv, xf, sx, wc, bc):
    M = xf.shape[0]
    BM = 2000

    def body(u, dv, x, s, w, b, o):
        d = dv[...]
        xb = x[...]
        ssx = -(d * (u[0] + u[1]))
        t2 = 2.0 * ssx - xb
        acc = jnp.dot(xb, w[0], preferred_element_type=_f32)
        acc += jnp.dot(s[...], w[1], preferred_element_type=_f32)
        acc += jnp.dot(t2, w[2], preferred_element_type=_f32)
        o[...] = acc + b[...]

    return pl.pallas_call(
        body,
        grid=(M // BM,),
        in_specs=[pl.BlockSpec((2, BM, CH), lambda i: (0, i, 0)),
                  pl.BlockSpec((BM, 1), lambda i: (i, 0)),
                  pl.BlockSpec((BM, CH), lambda i: (i, 0)),
                  pl.BlockSpec((BM, CH), lambda i: (i, 0)),
                  pl.BlockSpec((3, CH, 3 * CH), lambda i: (0, 0, 0)),
                  pl.BlockSpec((1, 3 * CH), lambda i: (0, 0))],
        out_specs=pl.BlockSpec((BM, 3 * CH), lambda i: (i, 0)),
        out_shape=jax.ShapeDtypeStruct((M, 3 * CH), _f32),
    )(u2p, dinv, xf, sx, wc, bc)


def _gate_call(a2p, dinv, xc, h, sh, wzr, bzr, t):
    BM = 2000
    ob = t * (N // BM)

    def body(u, dv, xcb, hb, shb, w, b, z_ref, hr_ref, w_ref):
        d = dv[...]
        hh = hb[...]
        ssh = -(d * (u[0] + u[1]))
        t2 = 2.0 * ssh - hh
        acc = jnp.dot(hh, w[0], preferred_element_type=_f32)
        acc += jnp.dot(shb[...], w[1], preferred_element_type=_f32)
        acc += jnp.dot(t2, w[2], preferred_element_type=_f32)
        g = jax.nn.sigmoid(xcb[...] + acc + b[...])
        z = g[:, :CH]
        hr = hh * g[:, CH:]
        z_ref[...] = z
        hr_ref[...] = hr
        w_ref[...] = d * hr

    return pl.pallas_call(
        body,
        grid=(N // BM,),
        in_specs=[pl.BlockSpec((2, BM, CH), lambda i: (0, i, 0)),
                  pl.BlockSpec((BM, 1), lambda i: (i + ob, 0)),
                  pl.BlockSpec((BM, 2 * CH), lambda i: (i + ob, 0)),
                  pl.BlockSpec((BM, CH), lambda i: (i, 0)),
                  pl.BlockSpec((BM, CH), lambda i: (i, 0)),
                  pl.BlockSpec((3, CH, 2 * CH), lambda i: (0, 0, 0)),
                  pl.BlockSpec((1, 2 * CH), lambda i: (0, 0))],
        out_specs=[pl.BlockSpec((BM, CH), lambda i: (i, 0))] * 3,
        out_shape=[jax.ShapeDtypeStruct((N, CH), _f32)] * 3,
    )(a2p, dinv, xc, h, sh, wzr, bzr)


def _update_call(b2p, dinv, xc, hr, shr, whh, bhh, z, h, t):
    BM = 2000
    ob = t * (N // BM)
    obn = min(t + 1, T - 1) * (N // BM)

    def body(u, dv, dvn, xcb, hrb, shrb, w, b, zb, hb, hn_ref, hd_ref):
        d = dv[...]
        hr_ = hrb[...]
        ss = -(d * (u[0] + u[1]))
        t2 = 2.0 * ss - hr_
        acc = jnp.dot(hr_, w[0], preferred_element_type=_f32)
        acc += jnp.dot(shrb[...], w[1], preferred_element_type=_f32)
        acc += jnp.dot(t2, w[2], preferred_element_type=_f32)
        ht = jnp.tanh(xcb[...] + acc + b[...])
        zz = zb[...]
        hn = zz * hb[...] + (1.0 - zz) * ht
        hn_ref[...] = hn
        hd_ref[...] = dvn[...] * hn

    return pl.pallas_call(
        body,
        grid=(N // BM,),
        in_specs=[pl.BlockSpec((2, BM, CH), lambda i: (0, i, 0)),
                  pl.BlockSpec((BM, 1), lambda i: (i + ob, 0)),
                  pl.BlockSpec((BM, 1), lambda i: (i + obn, 0)),
                  pl.BlockSpec((BM, CH), lambda i: (i + ob, 2)),
                  pl.BlockSpec((BM, CH), lambda i: (i, 0)),
                  pl.BlockSpec((BM, CH), lambda i: (i, 0)),
                  pl.BlockSpec((3, CH, CH), lambda i: (0, 0, 0)),
                  pl.BlockSpec((1, CH), lambda i: (0, 0)),
                  pl.BlockSpec((BM, CH), lambda i: (i, 0)),
                  pl.BlockSpec((BM, CH), lambda i: (i, 0))],
        out_specs=[pl.BlockSpec((BM, CH), lambda i: (i, 0))] * 2,
        out_shape=[jax.ShapeDtypeStruct((N, CH), _f32)] * 2,
    )(b2p, dinv, dinv, xc, hr, shr, whh, bhh, z, h)


def _proj_call(h, wp, bp):
    BM = 2000

    def body(hb, w, b, o):
        o[...] = jnp.dot(hb[...], w[...], preferred_element_type=_f32) + b[...]

    return pl.pallas_call(
        body,
        grid=(N // BM,),
        in_specs=[pl.BlockSpec((BM, CH), lambda i: (i, 0)),
                  pl.BlockSpec((CH, CH), lambda i: (0, 0)),
                  pl.BlockSpec((1, CH), lambda i: (0, 0))],
        out_specs=pl.BlockSpec((BM, CH), lambda i: (i, 0)),
        out_shape=jax.ShapeDtypeStruct((N, CH), _f32),
    )(h, wp, bp)


def kernel(features_seq, edges_seq, Wx, bx, Wh, bh, Wp, bp):
    i32 = jnp.int32
    src = edges_seq[:, 0]
    dst = edges_seq[:, 1]
    self_m = src == dst
    srcm = jnp.where(self_m, DUMMY, src).astype(i32)
    dstm = jnp.where(self_m, DUMMY, dst).astype(i32)
    toff = (jnp.arange(T, dtype=i32) * N)[:, None]
    srcx = (src + toff).astype(i32)

    def pad_idx(a, fill):
        a4 = a.reshape(T, NW, EPW)
        padcols = jnp.full((T, NW, SLOTS - EPW), fill, i32)
        return jnp.concatenate([a4, padcols], axis=2).reshape(T * NW * RPT8, QI)

    dstm2 = pad_idx(dstm, DUMMY)
    srcx2 = pad_idx(srcx, 0)
    srcr = pad_idx(src, 0).reshape(T, NW * RPT8, QI)
    dstmr = dstm2.reshape(T, NW * RPT8, QI)
    srcmd = jnp.transpose(pad_idx(srcm, DUMMY).reshape(T, NW, RPT8, QI),
                          (1, 0, 2, 3)).reshape(NW * T * RPT8, QI)

    zc128 = jnp.zeros((QI, CH), _f32)

    wc = jnp.transpose(Wx, (1, 2, 0, 3)).reshape(3, CH, 3 * CH)
    bc = bx.reshape(1, 3 * CH)
    wzr = jnp.transpose(Wh[:2], (1, 2, 0, 3)).reshape(3, CH, 2 * CH)
    bzr = bh[:2].reshape(1, 2 * CH)
    whh = Wh[2]
    bhh = bh[2].reshape(1, CH)
    bp2 = bp.reshape(1, CH)

    xf = features_seq.reshape(T * N, CH)

    pat = jnp.repeat(jnp.eye(T, dtype=_f32), CH // T, axis=1)
    patt = jnp.broadcast_to(pat[:, None, :], (T, QI, CH)).reshape(T * QI, CH)
    oh = jnp.eye(CH, dtype=_f32)[(CH // T) * jnp.arange(T)][:, :, None]
    degp = _deg_call(patt, srcmd, zc128).reshape(NC, N, CH)
    dinv, yhat = _prep_call(degp, oh, xf)

    u1 = _apply4(yhat, srcx2, dstm2, zc128).reshape(NC, T * N, CH)
    sx, v2 = _scale_call(u1, dinv)
    u2 = _apply4(v2, srcx2, dstm2, zc128).reshape(NC, T * N, CH)
    xc = _xmat_call(u2, dinv, xf, sx, wc, bc)

    zN = jnp.zeros((N, CH), _f32)
    z2 = jnp.zeros((NC, N, CH), _f32)
    H = zN
    Hd = zN
    for t in range(T):
        if t == 0:
            z, hr, wv = _gate_call(z2, dinv, xc, zN, zN, wzr, bzr, t)
            hn, hd = _update_call(z2, dinv, xc, zN, zN, whh, bhh, z, zN, t)
        else:
            a1 = _apply1(Hd, srcr[t], dstmr[t], zc128).reshape(NC, N, CH)
            sh, va = _scale_call(a1, dinv, toff=t)
            a2 = _apply1(va, srcr[t], dstmr[t], zc128).reshape(NC, N, CH)
            z, hr, wv = _gate_call(a2, dinv, xc, H, sh, wzr, bzr, t)
            b1 = _apply1(wv, srcr[t], dstmr[t], zc128).reshape(NC, N, CH)
            shr, vb = _scale_call(b1, dinv, toff=t)
            b2 = _apply1(vb, srcr[t], dstmr[t], zc128).reshape(NC, N, CH)
            hn, hd = _update_call(b2, dinv, xc, hr, shr, whh, bhh, z, H, t)
        H = hn
        Hd = hd
    return _proj_call(H, Wp, bp2)


# speedup vs baseline: 1.1862x; 1.0027x over previous
import functools

import jax
import jax.numpy as jnp
from jax import lax
from jax.experimental import pallas as pl
from jax.experimental.pallas import tpu as pltpu
from jax.experimental.pallas import tpu_sc as plsc

N = 10000
E = 320000
T = 4
CH = 128

NC = 2
NS = 16
NW = NC * NS
EPW = E // NW
QI = 64
SLOTS = 10240
RPT8 = SLOTS // QI
NPAD = 10240
DUMMY = N
ZPT = NPAD // NS
DRW = 624

_f32 = jnp.float32
_mesh = plsc.VectorSubcoreMesh(core_axis_name="c", subcore_axis_name="s")


def _make_apply(nt, rpt8, K=4, L=3, CHK=8):
    nchunk = rpt8 // CHK

    @functools.partial(
        pl.kernel,
        out_type=jax.ShapeDtypeStruct((NC * nt * N, CH), _f32),
        mesh=_mesh,
        scratch_types=[
            pltpu.VMEM_SHARED((NPAD, CH), _f32),
            pltpu.VMEM((CHK, QI), jnp.int32),
            pltpu.VMEM((CHK, QI), jnp.int32),
            pltpu.VMEM((CHK, QI), jnp.int32),
            pltpu.VMEM((CHK, QI), jnp.int32),
            pltpu.VMEM((QI, CH), _f32),
            pltpu.VMEM((QI, CH), _f32),
            pltpu.VMEM((QI, CH), _f32),
            pltpu.VMEM((QI, CH), _f32),
            pltpu.SemaphoreType.DMA,
            pltpu.SemaphoreType.DMA,
        ],
    )
    def apply_k(tab, srcg, dstm, zc, out, acc,
                srcb0, srcb1, dstb0, dstb1, r0, r1, r2, r3, gsem, ssem):
        cid = lax.axis_index("c")
        sid = lax.axis_index("s")
        wid = cid * NS + sid
        rows = (r0, r1, r2, r3)
        srcbs = (srcb0, srcb1)
        dstbs = (dstb0, dstb1)

        def scat(jj, gh, sh):
            k2 = jj % K
            gh[k2].wait()
            sh[k2] = pltpu.async_copy(
                rows[k2], acc.at[dstbs[(jj // CHK) % 2].at[jj % CHK]],
                ssem, add=True)

        def per_t(t, carry):
            pltpu.sync_copy(zc, r0)
            for c in range(ZPT // QI):
                pltpu.sync_copy(r0, acc.at[pl.ds(sid * ZPT + c * QI, QI)])
            plsc.subcore_barrier()
            gh = [None] * K
            sh = [None] * K
            for h in range(nchunk):
                b = h % 2
                slab = (t * NW + wid) * rpt8 + h * CHK
                pltpu.sync_copy(srcg.at[pl.ds(slab, CHK)], srcbs[b])
                pltpu.sync_copy(dstm.at[pl.ds(slab, CHK)], dstbs[b])
                for j8 in range(CHK):
                    j = h * CHK + j8
                    k = j % K
                    if sh[k] is not None:
                        sh[k].wait()
                    gh[k] = pltpu.async_copy(tab.at[srcbs[b].at[j8]],
                                             rows[k], gsem)
                    if j - L >= 0:
                        scat(j - L, gh, sh)
            for jj in range(rpt8 - L, rpt8):
                scat(jj, gh, sh)
            for k in range(K):
                if sh[k] is not None:
                    sh[k].wait()
            plsc.subcore_barrier()
            outbase = cid * (nt * N) + t * N
            pltpu.sync_copy(acc.at[pl.ds(sid * DRW, DRW)],
                            out.at[pl.ds(outbase + sid * DRW, DRW)])

            @pl.when(sid == NS - 1)
            def _():
                pltpu.sync_copy(acc.at[pl.ds(NS * DRW, N - NS * DRW)],
                                out.at[pl.ds(outbase + NS * DRW, N - NS * DRW)])

            plsc.subcore_barrier()
            return carry

        if nt == 1:
            per_t(0, 0)
        else:
            lax.fori_loop(0, nt, per_t, 0)

    return apply_k


_apply1 = _make_apply(1, rpt8=RPT8)
_apply4 = _make_apply(T, rpt8=RPT8)


@functools.partial(
    pl.kernel,
    out_type=jax.ShapeDtypeStruct((NC * N, CH), _f32),
    mesh=_mesh,
    scratch_types=[
        pltpu.VMEM_SHARED((NPAD, CH), _f32),
        pltpu.VMEM((16, QI), jnp.int32),
        pltpu.VMEM((16, QI), jnp.int32),
        pltpu.VMEM((QI, CH), _f32),
        pltpu.SemaphoreType.DMA,
    ],
)
def _deg_call(patt, dstm, zc, out, acc, dstb0, dstb1, rows, ssem):
    cid = lax.axis_index("c")
    sid = lax.axis_index("s")
    wid = cid * NS + sid
    pltpu.sync_copy(zc, rows)
    for c in range(ZPT // QI):
        pltpu.sync_copy(rows, acc.at[pl.ds(sid * ZPT + c * QI, QI)])
    plsc.subcore_barrier()
    dstbs = (dstb0, dstb1)
    pending = [[], []]
    for t in range(T):
        for b in (0, 1):
            for r in pending[b]:
                r.wait()
            pending[b] = []
        pltpu.sync_copy(patt.at[pl.ds(t * QI, QI)], rows)
        for h in range(RPT8 // 16):
            b = h % 2
            for r in pending[b]:
                r.wait()
            pending[b] = []
            pltpu.sync_copy(dstm.at[pl.ds((wid * T + t) * RPT8 + h * 16, 16)],
                            dstbs[b])
            for j in range(16):
                pending[b].append(pltpu.async_copy(rows, acc.at[dstbs[b].at[j]],
                                                   ssem, add=True))
    for b in (0, 1):
        for r in pending[b]:
            r.wait()
    plsc.subcore_barrier()
    outbase = cid * N
    pltpu.sync_copy(acc.at[pl.ds(sid * DRW, DRW)],
                    out.at[pl.ds(outbase + sid * DRW, DRW)])

    @pl.when(sid == NS - 1)
    def _():
        pltpu.sync_copy(acc.at[pl.ds(NS * DRW, N - NS * DRW)],
                        out.at[pl.ds(outbase + NS * DRW, N - NS * DRW)])


def _prep_call(degp, oh, xf):
    BM = 2000
    NB = N // BM

    def body(dp, o, x, dv, y):
        deg = jnp.dot(dp[0] + dp[1], o[0], preferred_element_type=_f32)
        d = jnp.where(deg > 0, lax.rsqrt(jnp.where(deg > 0, deg, 1.0)), 0.0)
        dv[...] = d
        y[...] = x[...] * d

    return pl.pallas_call(
        body,
        grid=(T, NB),
        in_specs=[pl.BlockSpec((2, BM, CH), lambda t, i: (0, i, 0)),
                  pl.BlockSpec((1, CH, 1), lambda t, i: (t, 0, 0)),
                  pl.BlockSpec((BM, CH), lambda t, i: (t * NB + i, 0))],
        out_specs=[pl.BlockSpec((BM, 1), lambda t, i: (t * NB + i, 0)),
                   pl.BlockSpec((BM, CH), lambda t, i: (t * NB + i, 0))],
        out_shape=[jax.ShapeDtypeStruct((T * N, 1), _f32),
                   jax.ShapeDtypeStruct((T * N, CH), _f32)],
    )(degp, oh, xf)


def _scale_call(up, dinv, toff=0):
    M = up.shape[1]
    BM = 2000
    ob = toff * (N // BM)

    def body(u, dv, s_ref, v_ref):
        d = dv[...]
        s = -(d * (u[0] + u[1]))
        s_ref[...] = s
        v_ref[...] = d * s

    return pl.pallas_call(
        body,
        grid=(M // BM,),
        in_specs=[pl.BlockSpec((2, BM, CH), lambda i: (0, i, 0)),
                  pl.BlockSpec((BM, 1), lambda i: (i + ob, 0))],
        out_specs=[pl.BlockSpec((BM, CH), lambda i: (i, 0)),
                   pl.BlockSpec((BM, CH), lambda i: (i, 0))],
        out_shape=[jax.ShapeDtypeStruct((M, CH), _f32),
                   jax.ShapeDtypeStruct((M, CH), _f32)],
    )(up, dinv)


def _xmat_call(u2p, dinv, xf, sx, wc, bc):
    M = xf.shape[0]
    BM = 2000

    def body(u, dv, x, s, w, b, o):
        d = dv[...]
        xb = x[...]
        ssx = -(d * (u[0] + u[1]))
        t2 = 2.0 * ssx - xb
        acc = jnp.dot(xb, w[0], preferred_element_type=_f32)
        acc += jnp.dot(s[...], w[1], preferred_element_type=_f32)
        acc += jnp.dot(t2, w[2], preferred_element_type=_f32)
        o[...] = acc + b[...]

    return pl.pallas_call(
        body,
        grid=(M // BM,),
        in_specs=[pl.BlockSpec((2, BM, CH), lambda i: (0, i, 0)),
                  pl.BlockSpec((BM, 1), lambda i: (i, 0)),
                  pl.BlockSpec((BM, CH), lambda i: (i, 0)),
                  pl.BlockSpec((BM, CH), lambda i: (i, 0)),
                  pl.BlockSpec((3, CH, 3 * CH), lambda i: (0, 0, 0)),
                  pl.BlockSpec((1, 3 * CH), lambda i: (0, 0))],
        out_specs=pl.BlockSpec((BM, 3 * CH), lambda i: (i, 0)),
        out_shape=jax.ShapeDtypeStruct((M, 3 * CH), _f32),
    )(u2p, dinv, xf, sx, wc, bc)


def _gate_call(a2p, dinv, xc, h, sh, wzr, bzr, t):
    BM = 2000
    ob = t * (N // BM)

    def body(u, dv, xcb, hb, shb, w, b, z_ref, hr_ref, w_ref):
        d = dv[...]
        hh = hb[...]
        ssh = -(d * (u[0] + u[1]))
        t2 = 2.0 * ssh - hh
        acc = jnp.dot(hh, w[0], preferred_element_type=_f32)
        acc += jnp.dot(shb[...], w[1], preferred_element_type=_f32)
        acc += jnp.dot(t2, w[2], preferred_element_type=_f32)
        g = jax.nn.sigmoid(xcb[...] + acc + b[...])
        z = g[:, :CH]
        hr = hh * g[:, CH:]
        z_ref[...] = z
        hr_ref[...] = hr
        w_ref[...] = d * hr

    return pl.pallas_call(
        body,
        grid=(N // BM,),
        in_specs=[pl.BlockSpec((2, BM, CH), lambda i: (0, i, 0)),
                  pl.BlockSpec((BM, 1), lambda i: (i + ob, 0)),
                  pl.BlockSpec((BM, 2 * CH), lambda i: (i + ob, 0)),
                  pl.BlockSpec((BM, CH), lambda i: (i, 0)),
                  pl.BlockSpec((BM, CH), lambda i: (i, 0)),
                  pl.BlockSpec((3, CH, 2 * CH), lambda i: (0, 0, 0)),
                  pl.BlockSpec((1, 2 * CH), lambda i: (0, 0))],
        out_specs=[pl.BlockSpec((BM, CH), lambda i: (i, 0))] * 3,
        out_shape=[jax.ShapeDtypeStruct((N, CH), _f32)] * 3,
    )(a2p, dinv, xc, h, sh, wzr, bzr)


def _update_call(b2p, dinv, xc, hr, shr, whh, bhh, z, h, t):
    BM = 2000
    ob = t * (N // BM)
    obn = min(t + 1, T - 1) * (N // BM)

    def body(u, dv, dvn, xcb, hrb, shrb, w, b, zb, hb, hn_ref, hd_ref):
        d = dv[...]
        hr_ = hrb[...]
        ss = -(d * (u[0] + u[1]))
        t2 = 2.0 * ss - hr_
        acc = jnp.dot(hr_, w[0], preferred_element_type=_f32)
        acc += jnp.dot(shrb[...], w[1], preferred_element_type=_f32)
        acc += jnp.dot(t2, w[2], preferred_element_type=_f32)
        ht = jnp.tanh(xcb[...] + acc + b[...])
        zz = zb[...]
        hn = zz * hb[...] + (1.0 - zz) * ht
        hn_ref[...] = hn
        hd_ref[...] = dvn[...] * hn

    return pl.pallas_call(
        body,
        grid=(N // BM,),
        in_specs=[pl.BlockSpec((2, BM, CH), lambda i: (0, i, 0)),
                  pl.BlockSpec((BM, 1), lambda i: (i + ob, 0)),
                  pl.BlockSpec((BM, 1), lambda i: (i + obn, 0)),
                  pl.BlockSpec((BM, CH), lambda i: (i + ob, 2)),
                  pl.BlockSpec((BM, CH), lambda i: (i, 0)),
                  pl.BlockSpec((BM, CH), lambda i: (i, 0)),
                  pl.BlockSpec((3, CH, CH), lambda i: (0, 0, 0)),
                  pl.BlockSpec((1, CH), lambda i: (0, 0)),
                  pl.BlockSpec((BM, CH), lambda i: (i, 0)),
                  pl.BlockSpec((BM, CH), lambda i: (i, 0))],
        out_specs=[pl.BlockSpec((BM, CH), lambda i: (i, 0))] * 2,
        out_shape=[jax.ShapeDtypeStruct((N, CH), _f32)] * 2,
    )(b2p, dinv, dinv, xc, hr, shr, whh, bhh, z, h)


def _proj_call(h, wp, bp):
    BM = 2000

    def body(hb, w, b, o):
        o[...] = jnp.dot(hb[...], w[...], preferred_element_type=_f32) + b[...]

    return pl.pallas_call(
        body,
        grid=(N // BM,),
        in_specs=[pl.BlockSpec((BM, CH), lambda i: (i, 0)),
                  pl.BlockSpec((CH, CH), lambda i: (0, 0)),
                  pl.BlockSpec((1, CH), lambda i: (0, 0))],
        out_specs=pl.BlockSpec((BM, CH), lambda i: (i, 0)),
        out_shape=jax.ShapeDtypeStruct((N, CH), _f32),
    )(h, wp, bp)


def kernel(features_seq, edges_seq, Wx, bx, Wh, bh, Wp, bp):
    i32 = jnp.int32
    src = edges_seq[:, 0]
    dst = edges_seq[:, 1]
    self_m = src == dst
    srcm = jnp.where(self_m, DUMMY, src).astype(i32)
    dstm = jnp.where(self_m, DUMMY, dst).astype(i32)
    toff = (jnp.arange(T, dtype=i32) * N)[:, None]
    srcx = (src + toff).astype(i32)

    def pad_idx(a, fill):
        a4 = a.reshape(T, NW, EPW)
        padcols = jnp.full((T, NW, SLOTS - EPW), fill, i32)
        return jnp.concatenate([a4, padcols], axis=2).reshape(T * NW * RPT8, QI)

    dstm2 = pad_idx(dstm, DUMMY)
    srcx2 = pad_idx(srcx, 0)
    srcr = pad_idx(src, 0).reshape(T, NW * RPT8, QI)
    dstmr = dstm2.reshape(T, NW * RPT8, QI)
    srcmd = jnp.transpose(pad_idx(srcm, DUMMY).reshape(T, NW, RPT8, QI),
                          (1, 0, 2, 3)).reshape(NW * T * RPT8, QI)

    zc128 = jnp.zeros((QI, CH), _f32)

    wc = jnp.transpose(Wx, (1, 2, 0, 3)).reshape(3, CH, 3 * CH)
    bc = bx.reshape(1, 3 * CH)
    wzr = jnp.transpose(Wh[:2], (1, 2, 0, 3)).reshape(3, CH, 2 * CH)
    bzr = bh[:2].reshape(1, 2 * CH)
    whh = Wh[2]
    bhh = bh[2].reshape(1, CH)
    bp2 = bp.reshape(1, CH)

    xf = features_seq.reshape(T * N, CH)

    pat = jnp.repeat(jnp.eye(T, dtype=_f32), CH // T, axis=1)
    patt = jnp.broadcast_to(pat[:, None, :], (T, QI, CH)).reshape(T * QI, CH)
    oh = jnp.eye(CH, dtype=_f32)[(CH // T) * jnp.arange(T)][:, :, None]
    degp = _deg_call(patt, srcmd, zc128).reshape(NC, N, CH)
    dinv, yhat = _prep_call(degp, oh, xf)

    u1 = _apply4(yhat, srcx2, dstm2, zc128).reshape(NC, T * N, CH)
    sx, v2 = _scale_call(u1, dinv)
    u2 = _apply4(v2, srcx2, dstm2, zc128).reshape(NC, T * N, CH)
    xc = _xmat_call(u2, dinv, xf, sx, wc, bc)

    zN = jnp.zeros((N, CH), _f32)
    z2 = jnp.zeros((NC, N, CH), _f32)
    H = zN
    Hd = zN
    for t in range(T):
        if t == 0:
            z, hr, wv = _gate_call(z2, dinv, xc, zN, zN, wzr, bzr, t)
            hn, hd = _update_call(z2, dinv, xc, zN, zN, whh, bhh, z, zN, t)
        else:
            a1 = _apply1(Hd, srcr[t], dstmr[t], zc128).reshape(NC, N, CH)
            sh, va = _scale_call(a1, dinv, toff=t)
            a2 = _apply1(va, srcr[t], dstmr[t], zc128).reshape(NC, N, CH)
            z, hr, wv = _gate_call(a2, dinv, xc, H, sh, wzr, bzr, t)
            b1 = _apply1(wv, srcr[t], dstmr[t], zc128).reshape(NC, N, CH)
            shr, vb = _scale_call(b1, dinv, toff=t)
            b2 = _apply1(vb, srcr[t], dstmr[t], zc128).reshape(NC, N, CH)
            hn, hd = _update_call(b2, dinv, xc, hr, shr, whh, bhh, z, H, t)
        H = hn
        Hd = hd
    return _proj_call(H, Wp, bp2)
